# baseline (speedup 1.0000x reference)
import functools

import jax
import jax.numpy as jnp
from jax import lax
from jax.experimental import pallas as pl
from jax.experimental.pallas import tpu as pltpu
from jax.experimental.pallas import tpu_sc as plsc

BATCH = 16384
HID = 64
HID2 = 2 * HID
N_LOOKUPS = 3
TOTAL = N_LOOKUPS * BATCH
TOTAL2 = 2 * TOTAL
NPAIR = 500000

_INFO = plsc.get_sparse_core_info()
_NC = _INFO.num_cores
_NS = _INFO.num_subcores
_NW = _NC * _NS
_CHUNK = 128
_B_PER_W = TOTAL2 // _NW
_N_CHUNKS = _B_PER_W // _CHUNK
_CH_IT = 3
_ROWS_IT = _CH_IT * _CHUNK
_N_IT = _N_CHUNKS // _CH_IT


def _make_gather():
    mesh = plsc.VectorSubcoreMesh(core_axis_name="c", subcore_axis_name="s")

    @functools.partial(
        pl.kernel,
        mesh=mesh,
        compiler_params=pltpu.CompilerParams(use_tc_tiling_on_sc=True),
        out_type=jax.ShapeDtypeStruct((TOTAL2, HID2), jnp.float32),
        scratch_types=[
            pltpu.VMEM((_N_CHUNKS, _CHUNK), jnp.int32),
            pltpu.VMEM((_ROWS_IT, HID2), jnp.float32),
            pltpu.VMEM((_ROWS_IT, HID2), jnp.float32),
            pltpu.SemaphoreType.DMA,
            pltpu.SemaphoreType.DMA,
            pltpu.SemaphoreType.DMA,
            pltpu.SemaphoreType.DMA,
        ],
    )
    def gather_kernel(table_hbm, idx_hbm, out_hbm, idx_v, rows_a, rows_b,
                      sg_a, sg_b, sw_a, sw_b):
        wid = lax.axis_index("s") * _NC + lax.axis_index("c")
        base = wid * _B_PER_W
        bufs = (rows_a, rows_b)
        sg = (sg_a, sg_b)
        sw = (sw_a, sw_b)
        pltpu.sync_copy(idx_hbm.at[wid], idx_v)
        writes = [None, None]
        for h in range(_N_IT):
            p = h % 2
            if writes[p] is not None:
                writes[p].wait()
            gs = []
            for j in range(_CH_IT):
                gs.append(
                    pltpu.async_copy(
                        table_hbm.at[idx_v.at[h * _CH_IT + j]],
                        bufs[p].at[pl.ds(j * _CHUNK, _CHUNK)],
                        sg[p],
                    )
                )
            for g in gs:
                g.wait()
            writes[p] = pltpu.async_copy(
                bufs[p],
                out_hbm.at[pl.ds(base + h * _ROWS_IT, _ROWS_IT)],
                sw[p],
            )
        writes[0].wait()
        writes[1].wait()

    return gather_kernel


_gather = _make_gather()

_NB = 8
_BM = BATCH // _NB


def _sel(g_ref, par_ref):
    g = g_ref[...]
    par = par_ref[...]
    return jnp.where(par > 0.5, g[:, HID:HID2], g[:, 0:HID])


def _dense_body(p0_ref, p1_ref, p2_ref, r_ref, i_ref, u_ref,
                q0_ref, q1_ref, q2_ref, qr_ref, qi_ref, qu_ref,
                w_ref, b_ref, eye_ref, p_ref, h_ref):
    eye = eye_ref[...]
    dn = (((1,), (1,)), ((), ()))
    hi = lax.Precision.HIGHEST
    for j, (ref, qref) in enumerate(
            ((p0_ref, q0_ref), (p1_ref, q1_ref), (p2_ref, q2_ref))):
        p_ref[j] = lax.dot_general(eye, _sel(ref, qref), dn, precision=hi,
                                   preferred_element_type=jnp.float32)
    w = w_ref[...]
    acc = lax.dot_general(w[:, 0:HID], _sel(r_ref, qr_ref), dn,
                          precision=hi, preferred_element_type=jnp.float32)
    acc += lax.dot_general(w[:, HID:2 * HID], _sel(i_ref, qi_ref), dn,
                           precision=hi, preferred_element_type=jnp.float32)
    acc += lax.dot_general(w[:, 2 * HID:3 * HID], _sel(u_ref, qu_ref), dn,
                           precision=hi, preferred_element_type=jnp.float32)
    h_ref[...] = jnp.tanh(acc + b_ref[...])


def kernel(userId, itemId, rate, table, W, b):
    flat_ctx = jnp.concatenate(
        [rate.astype(jnp.int32), itemId.astype(jnp.int32),
         userId.astype(jnp.int32)]
    )
    flat_p = flat_ctx.reshape(BATCH, N_LOOKUPS).T.reshape(TOTAL)
    idx_all = jnp.concatenate([flat_ctx, flat_p])
    pair_idx = lax.shift_right_logical(idx_all, 1).reshape(
        _NW, _N_CHUNKS, _CHUNK)
    parity = lax.convert_element_type(
        jnp.bitwise_and(idx_all, 1), jnp.float32).reshape(TOTAL2, 1)

    table_pairs = table.reshape(NPAIR, HID2)
    gathered = _gather(table_pairs, pair_idx)

    nb3 = N_LOOKUPS * _NB
    g_specs = [
        pl.BlockSpec((_BM, HID2), lambda i, o=o: (i + o, 0))
        for o in (nb3, nb3 + _NB, nb3 + 2 * _NB, 0, _NB, 2 * _NB)
    ]
    q_specs = [
        pl.BlockSpec((_BM, 1), lambda i, o=o: (i + o, 0))
        for o in (nb3, nb3 + _NB, nb3 + 2 * _NB, 0, _NB, 2 * _NB)
    ]
    P, H = pl.pallas_call(
        _dense_body,
        grid=(_NB,),
        in_specs=g_specs + q_specs + [
            pl.BlockSpec((HID, N_LOOKUPS * HID), lambda i: (0, 0)),
            pl.BlockSpec((HID, 1), lambda i: (0, 0)),
            pl.BlockSpec((HID, HID), lambda i: (0, 0)),
        ],
        out_specs=[
            pl.BlockSpec((N_LOOKUPS, HID, _BM), lambda i: (0, 0, i)),
            pl.BlockSpec((HID, _BM), lambda i: (0, i)),
        ],
        out_shape=[
            jax.ShapeDtypeStruct((N_LOOKUPS, HID, BATCH), jnp.float32),
            jax.ShapeDtypeStruct((HID, BATCH), jnp.float32),
        ],
    )(gathered, gathered, gathered, gathered, gathered, gathered,
      parity, parity, parity, parity, parity, parity,
      W, b.reshape(HID, 1), jnp.eye(HID, dtype=jnp.float32))

    output_list = jnp.transpose(P, (2, 0, 1))
    hidden = H.T
    return (output_list, hidden)
